# Initial kernel scaffold; baseline (speedup 1.0000x reference)
import jax
import jax.numpy as jnp
from jax.experimental import pallas as pl


def kernel(x, Wq, bq, Wk, bk, Wv, bv, Wo, bo, proj):
    raise NotImplementedError("write your pallas kernel here")


# speedup vs baseline: 5.7355x; 5.7355x over previous
import functools

import jax
import jax.numpy as jnp
from jax.experimental import pallas as pl
from jax.experimental.pallas import tpu as pltpu

NUM_HEADS = 16
HEAD_DIM = 128
HIDDEN = 2048
NUM_HASHES = 8
SCALE = HEAD_DIM ** (-0.5)
B, S = 2, 4096
GROUP = 8


def _qkv_body(x_ref, w_ref, b_ref, out_ref):
    acc = jnp.dot(x_ref[...], w_ref[...], preferred_element_type=jnp.float32)
    acc = acc + b_ref[...]
    bm = acc.shape[0]
    out_ref[...] = acc.reshape(bm, -1, HEAD_DIM).swapaxes(0, 1)[None]


def _qkv_proj(x2d, wqkv, bqkv, bm, bn):
    m_tiles = x2d.shape[0] // bm
    n_tiles = wqkv.shape[1] // bn
    heads_per_n = bn // HEAD_DIM
    s_tiles = S // bm
    out = pl.pallas_call(
        _qkv_body,
        grid=(n_tiles, m_tiles),
        in_specs=[
            pl.BlockSpec((bm, HIDDEN), lambda n, m: (m, 0)),
            pl.BlockSpec((HIDDEN, bn), lambda n, m: (0, n)),
            pl.BlockSpec((1, bn), lambda n, m: (0, n)),
        ],
        out_specs=pl.BlockSpec(
            (1, heads_per_n, bm, HEAD_DIM),
            lambda n, m: (m // s_tiles, n, m % s_tiles, 0),
        ),
        out_shape=jax.ShapeDtypeStruct((B, 3 * NUM_HEADS, S, HEAD_DIM),
                                       jnp.float32),
    )(x2d, wqkv, bqkv.reshape(1, -1))
    return out


def _bucket_body(q_ref, projt_ref, out_ref):
    q = q_ref[0, 0]
    qp = jnp.dot(q, projt_ref[...], preferred_element_type=jnp.float32)
    bits = (qp > 0).astype(jnp.int32)
    powers = (2 ** jax.lax.iota(jnp.int32, NUM_HASHES))[None, :]
    out_ref[0, 0, 0] = jnp.sum(bits * powers, axis=-1)


def _lsh_buckets(qkv, proj):
    projt = proj.T
    out = pl.pallas_call(
        _bucket_body,
        grid=(B, NUM_HEADS),
        in_specs=[
            pl.BlockSpec((1, 1, S, HEAD_DIM), lambda b, h: (b, h, 0, 0)),
            pl.BlockSpec((HEAD_DIM, NUM_HASHES), lambda b, h: (0, 0)),
        ],
        out_specs=pl.BlockSpec((1, 1, 1, S), lambda b, h: (b, h, 0, 0)),
        out_shape=jax.ShapeDtypeStruct((B, NUM_HEADS, 1, S), jnp.int32),
    )(qkv, projt)
    return out.reshape(B, NUM_HEADS, S)


def _attn_body(q_ref, k_ref, v_ref, wo_ref, bo_ref, out_ref, att_ref):
    n_groups = q_ref.shape[0]

    r = jax.lax.broadcasted_iota(jnp.int32, (GROUP * NUM_HEADS,
                                             GROUP * NUM_HEADS), 0)
    c = jax.lax.broadcasted_iota(jnp.int32, (GROUP * NUM_HEADS,
                                             GROUP * NUM_HEADS), 1)
    same_pos = (r // NUM_HEADS) == (c // NUM_HEADS)

    for g in range(n_groups):
        qg, kg, vg = q_ref[g], k_ref[g], v_ref[g]
        s = jax.lax.dot_general(
            qg, kg, (((1,), (1,)), ((), ())),
            preferred_element_type=jnp.float32) * SCALE
        s = jnp.where(same_pos, s, -jnp.inf)
        s = s - jnp.max(s, axis=-1, keepdims=True)
        e = jnp.exp(s)
        p = e / jnp.sum(e, axis=-1, keepdims=True)
        og = jnp.dot(p, vg, preferred_element_type=jnp.float32)
        att_ref[g * GROUP:(g + 1) * GROUP, :] = og.reshape(GROUP, HIDDEN)

    out_ref[...] = (
        jnp.dot(att_ref[...], wo_ref[...], preferred_element_type=jnp.float32)
        + bo_ref[...])


def _attn_proj(q_s, k_s, v_s, wo, bo, bm):
    m_tiles = (B * S) // bm
    g = bm // GROUP
    out = pl.pallas_call(
        _attn_body,
        grid=(m_tiles,),
        in_specs=[
            pl.BlockSpec((g, GROUP * NUM_HEADS, HEAD_DIM),
                         lambda m: (m, 0, 0)),
            pl.BlockSpec((g, GROUP * NUM_HEADS, HEAD_DIM),
                         lambda m: (m, 0, 0)),
            pl.BlockSpec((g, GROUP * NUM_HEADS, HEAD_DIM),
                         lambda m: (m, 0, 0)),
            pl.BlockSpec((HIDDEN, HIDDEN), lambda m: (0, 0)),
            pl.BlockSpec((1, HIDDEN), lambda m: (0, 0)),
        ],
        out_specs=pl.BlockSpec((bm, HIDDEN), lambda m: (m, 0)),
        out_shape=jax.ShapeDtypeStruct((B * S, HIDDEN), jnp.float32),
        scratch_shapes=[pltpu.VMEM((bm, HIDDEN), jnp.float32)],
    )(q_s, k_s, v_s, wo, bo.reshape(1, -1))
    return out


@jax.jit
def kernel(x, Wq, bq, Wk, bk, Wv, bv, Wo, bo, proj):
    x2d = x.reshape(B * S, HIDDEN)
    wqkv = jnp.concatenate([Wq, Wk, Wv], axis=1)
    bqkv = jnp.concatenate([bq, bk, bv])

    qkv = _qkv_proj(x2d, wqkv, bqkv, bm=512, bn=1024)
    buckets = _lsh_buckets(qkv, proj)

    idx = jnp.argsort(buckets, axis=2, stable=True)
    def take(part):
        t = qkv[:, part * NUM_HEADS:(part + 1) * NUM_HEADS]
        g = jnp.take_along_axis(t, idx[..., None], axis=2)
        return g.swapaxes(1, 2).reshape(
            B * S // GROUP, GROUP * NUM_HEADS, HEAD_DIM)

    q_s, k_s, v_s = take(0), take(1), take(2)

    out = _attn_proj(q_s, k_s, v_s, Wo, bo, bm=256)
    return out.reshape(B, S, HIDDEN)


# speedup vs baseline: 8.2567x; 1.4396x over previous
import functools

import jax
import jax.numpy as jnp
from jax import lax
from jax.experimental import pallas as pl
from jax.experimental.pallas import tpu as pltpu
from jax.experimental.pallas import tpu_sc as plsc

NUM_HEADS = 16
HEAD_DIM = 128
HIDDEN = 2048
NUM_HASHES = 8
SCALE = HEAD_DIM ** (-0.5)
B, S = 2, 4096
GROUP = 8
ROW = 3 * HEAD_DIM
CHUNK = 128
N_CHUNKS = S // CHUNK


def _qkv_body(x_ref, w_ref, b_ref, out_ref):
    acc = jnp.dot(x_ref[...], w_ref[...], preferred_element_type=jnp.float32)
    acc = acc + b_ref[...]
    bm = acc.shape[0]
    out_ref[...] = acc.reshape(bm, -1, HEAD_DIM).swapaxes(0, 1)[None]


def _qkv_proj(x2d, wqkv, bqkv, bm, bn):
    m_tiles = x2d.shape[0] // bm
    n_tiles = wqkv.shape[1] // bn
    heads_per_n = bn // HEAD_DIM
    n_per_part = HIDDEN // bn
    s_tiles = S // bm
    return pl.pallas_call(
        _qkv_body,
        grid=(n_tiles, m_tiles),
        in_specs=[
            pl.BlockSpec((bm, HIDDEN), lambda n, m: (m, 0)),
            pl.BlockSpec((HIDDEN, bn), lambda n, m: (0, n)),
            pl.BlockSpec((1, bn), lambda n, m: (0, n)),
        ],
        out_specs=pl.BlockSpec(
            (1, heads_per_n, bm, HEAD_DIM),
            lambda n, m: (m // s_tiles, n % n_per_part, m % s_tiles,
                          n // n_per_part),
        ),
        out_shape=jax.ShapeDtypeStruct((B, NUM_HEADS, S, ROW), jnp.float32),
    )(x2d, wqkv, bqkv.reshape(1, -1))


def _bucket_body(q_ref, projt_ref, out_ref):
    q = q_ref[0, 0]
    qp = jnp.dot(q, projt_ref[...], preferred_element_type=jnp.float32)
    bits = (qp > 0).astype(jnp.int32)
    powers = (2 ** lax.iota(jnp.int32, NUM_HASHES))[None, :]
    out_ref[0, 0, 0] = jnp.sum(bits * powers, axis=-1)


def _lsh_buckets(qkv, proj):
    out = pl.pallas_call(
        _bucket_body,
        grid=(B, NUM_HEADS),
        in_specs=[
            pl.BlockSpec((1, 1, S, HEAD_DIM), lambda b, h: (b, h, 0, 0)),
            pl.BlockSpec((HEAD_DIM, NUM_HASHES), lambda b, h: (0, 0)),
        ],
        out_specs=pl.BlockSpec((1, 1, 1, S), lambda b, h: (b, h, 0, 0)),
        out_shape=jax.ShapeDtypeStruct((B, NUM_HEADS, 1, S), jnp.int32),
    )(qkv, proj.T)
    return out.reshape(B * NUM_HEADS * S)


def _sc_sort_gather(buckets_flat, qkv_table):
    mesh = plsc.VectorSubcoreMesh(core_axis_name="c", subcore_axis_name="s")

    @functools.partial(
        pl.kernel,
        out_type=jax.ShapeDtypeStruct((B * S * NUM_HEADS, ROW), jnp.float32),
        mesh=mesh,
        scratch_types=[
            pltpu.VMEM((S,), jnp.int32),
            pltpu.VMEM((16 * 256,), jnp.int32),
            pltpu.VMEM((S,), jnp.int32),
            pltpu.VMEM((N_CHUNKS, CHUNK), jnp.int32),
            pltpu.VMEM((CHUNK, ROW), jnp.float32),
            pltpu.VMEM((CHUNK, ROW), jnp.float32),
            pltpu.SMEM((256,), jnp.int32),
            pltpu.SemaphoreType.DMA,
            pltpu.SemaphoreType.DMA,
            pltpu.SemaphoreType.DMA,
            pltpu.SemaphoreType.DMA,
        ],
        compiler_params=pltpu.CompilerParams(needs_layout_passes=False),
    )
    def sc_kernel(buckets_hbm, qkv_hbm, out_hbm, keys, hist2d, gidx,
                  sidx, rows0, rows1, offs, g0, g1, s0, s1):
        w = lax.axis_index("s") * 2 + lax.axis_index("c")
        b = w // NUM_HEADS
        h = w % NUM_HEADS
        src_base = w * S
        dst_base = b * S * NUM_HEADS + h

        lane = lax.iota(jnp.int32, 16)
        zero16 = jnp.zeros((16,), jnp.int32)
        ones16 = jnp.ones((16,), jnp.int32)

        pltpu.sync_copy(buckets_hbm.at[pl.ds(w * S, S)], keys)

        for j in range(16 * 256 // 16):
            hist2d[pl.ds(j * 16, 16)] = zero16

        lane256 = lane * 256
        def hist_body(c, carry):
            k16 = keys[pl.ds(c * 16, 16)]
            slot = lane256 + k16
            cnt = plsc.load_gather(hist2d, [slot])
            plsc.store_scatter(hist2d, [slot], cnt + ones16)
            return carry
        lax.fori_loop(0, S // 16, hist_body, 0, unroll=4)

        carry_in = jnp.int32(0)
        for g in range(16):
            tot = zero16
            for l in range(16):
                tot = tot + hist2d[pl.ds(l * 256 + g * 16, 16)]
            incl = plsc.cumsum(tot)
            excl = incl - tot + carry_in
            for l in range(16):
                offs[g * 16 + l] = excl[l]
            carry_in = carry_in + incl[15]

        def place_body(c, carry):
            k16 = keys[pl.ds(c * 16, 16)]
            src16 = src_base + c * 16 + lane
            rvec = zero16
            for l in range(16):
                k = k16[l]
                r = offs[k]
                offs[k] = r + 1
                rvec = jnp.where(lane == l, r, rvec)
            plsc.store_scatter(gidx, [rvec], src16)
            return carry
        lax.fori_loop(0, S // 16, place_body, 0)

        for c in range(N_CHUNKS):
            for g in range(CHUNK // 16):
                t0 = c * CHUNK + g * 16
                sidx[c, pl.ds(g * 16, 16)] = (
                    dst_base + (t0 + lane) * NUM_HEADS)

        bufs = (rows0, rows1)
        gsems = (g0, g1)
        ssems = (s0, s1)

        def chunk_step(c, p):
            buf, gs, ss = bufs[p], gsems[p], ssems[p]

            @pl.when(c >= 2)
            def _():
                pltpu.make_async_copy(buf, out_hbm.at[sidx.at[c - 2]],
                                      ss).wait()

            pltpu.make_async_copy(
                qkv_hbm.at[gidx.at[pl.ds(c * CHUNK, CHUNK)]], buf, gs).start()
            pltpu.make_async_copy(
                qkv_hbm.at[gidx.at[pl.ds(c * CHUNK, CHUNK)]], buf, gs).wait()
            pltpu.make_async_copy(buf, out_hbm.at[sidx.at[c]], ss).start()

        def outer(c, carry):
            chunk_step(c * 2, 0)
            chunk_step(c * 2 + 1, 1)
            return carry
        lax.fori_loop(0, N_CHUNKS // 2, outer, 0)

        for p in range(2):
            pltpu.make_async_copy(
                bufs[p], out_hbm.at[sidx.at[N_CHUNKS - 2 + p]],
                ssems[p]).wait()

    return sc_kernel(buckets_flat, qkv_table)


def _attn_body(qkv_ref, wo_ref, bo_ref, out_ref, att_ref):
    n_groups = qkv_ref.shape[0]

    r = lax.broadcasted_iota(jnp.int32, (GROUP * NUM_HEADS,
                                         GROUP * NUM_HEADS), 0)
    c = lax.broadcasted_iota(jnp.int32, (GROUP * NUM_HEADS,
                                         GROUP * NUM_HEADS), 1)
    same_pos = (r // NUM_HEADS) == (c // NUM_HEADS)

    for g in range(n_groups):
        blk = qkv_ref[g]
        qg = blk[:, :HEAD_DIM]
        kg = blk[:, HEAD_DIM:2 * HEAD_DIM]
        vg = blk[:, 2 * HEAD_DIM:]
        s = lax.dot_general(
            qg, kg, (((1,), (1,)), ((), ())),
            preferred_element_type=jnp.float32) * SCALE
        s = jnp.where(same_pos, s, -jnp.inf)
        s = s - jnp.max(s, axis=-1, keepdims=True)
        e = jnp.exp(s)
        p = e / jnp.sum(e, axis=-1, keepdims=True)
        og = jnp.dot(p, vg, preferred_element_type=jnp.float32)
        att_ref[g * GROUP:(g + 1) * GROUP, :] = og.reshape(GROUP, HIDDEN)

    out_ref[...] = (
        jnp.dot(att_ref[...], wo_ref[...], preferred_element_type=jnp.float32)
        + bo_ref[...])


def _attn_proj(qkv_s, wo, bo, bm):
    m_tiles = (B * S) // bm
    g = bm // GROUP
    return pl.pallas_call(
        _attn_body,
        grid=(m_tiles,),
        in_specs=[
            pl.BlockSpec((g, GROUP * NUM_HEADS, ROW), lambda m: (m, 0, 0)),
            pl.BlockSpec((HIDDEN, HIDDEN), lambda m: (0, 0)),
            pl.BlockSpec((1, HIDDEN), lambda m: (0, 0)),
        ],
        out_specs=pl.BlockSpec((bm, HIDDEN), lambda m: (m, 0)),
        out_shape=jax.ShapeDtypeStruct((B * S, HIDDEN), jnp.float32),
        scratch_shapes=[pltpu.VMEM((bm, HIDDEN), jnp.float32)],
    )(qkv_s, wo, bo.reshape(1, -1))


@jax.jit
def kernel(x, Wq, bq, Wk, bk, Wv, bv, Wo, bo, proj):
    x2d = x.reshape(B * S, HIDDEN)
    wqkv = jnp.concatenate([Wq, Wk, Wv], axis=1)
    bqkv = jnp.concatenate([bq, bk, bv])

    qkv = _qkv_proj(x2d, wqkv, bqkv, bm=512, bn=1024)
    buckets = _lsh_buckets(qkv, proj)

    qkv_s = _sc_sort_gather(buckets, qkv.reshape(B * NUM_HEADS * S, ROW))
    qkv_s = qkv_s.reshape(B * S // GROUP, GROUP * NUM_HEADS, ROW)

    out = _attn_proj(qkv_s, Wo, bo, bm=256)
    return out.reshape(B, S, HIDDEN)
